# baseline (speedup 1.0000x reference)
import functools

import jax
import jax.numpy as jnp
from jax import lax
from jax.experimental import pallas as pl
from jax.experimental.pallas import tpu as pltpu
from jax.experimental.pallas import tpu_sc as plsc
from jax._src.pallas import mpmd as _mpmd


@functools.lru_cache(maxsize=None)
def _make_fifo(capacity: int, n: int, dims: int):
  info = plsc.get_sparse_core_info()
  nc, ns, lanes = info.num_cores, info.num_subcores, info.num_lanes
  nw = nc * ns
  assert capacity % nw == 0 and n % ns == 0 and n % lanes == 0
  assert n <= capacity // nc
  slab = capacity // nw
  piece = n // ns
  half = capacity // nc
  mesh = plsc.VectorSubcoreMesh(core_axis_name="c", subcore_axis_name="s")

  def body(storage_ref, vals_ref, ptr_ref, out_ref,
           didx_v, vidx_v, rows_v, ptr_v, sem_c, sem_g, sem_s):
    c = lax.axis_index("c")
    s = lax.axis_index("s")
    wid = c * ns + s
    copy = pltpu.async_copy(
        storage_ref.at[pl.ds(wid * slab, slab)],
        out_ref.at[pl.ds(wid * slab, slab)],
        sem_c,
    )

    pltpu.sync_copy(ptr_ref, ptr_v)
    p = lax.rem(ptr_v[...][0], capacity)
    wrap = jnp.maximum(p + n - capacity, 0)
    a_end = p + n - wrap
    h0 = c * half
    h1 = h0 + half
    lo_a = jnp.maximum(p, h0)
    len_a = jnp.maximum(jnp.minimum(a_end, h1) - lo_a, 0)
    len_b = jnp.maximum(jnp.minimum(wrap, h1) - h0, 0)
    win_len = len_a + len_b
    lo = jnp.where(len_a > 0, lo_a, h0)
    vbase = lax.rem(lo - p + capacity, capacity)

    off = s * piece
    @pl.when(off < win_len)
    def _gather():
      last = win_len - 1
      for i in range(piece // lanes):
        m = jnp.minimum(off + i * lanes + lax.iota(jnp.int32, lanes), last)
        didx_v[pl.ds(i * lanes, lanes)] = lo + m
        vidx_v[pl.ds(i * lanes, lanes)] = vbase + m
      pltpu.async_copy(vals_ref.at[vidx_v], rows_v, sem_g).wait()

    copy.wait()
    plsc.subcore_barrier()

    @pl.when(off < win_len)
    def _scatter():
      pltpu.async_copy(rows_v, out_ref.at[didx_v], sem_s).wait()

  return _mpmd.mpmd_map(
      [(mesh, body)],
      out_types=jax.ShapeDtypeStruct((capacity, dims), jnp.float32),
      scratch_types=[
          pltpu.VMEM((piece,), jnp.int32),
          pltpu.VMEM((piece,), jnp.int32),
          pltpu.VMEM((piece, dims), jnp.float32),
          pltpu.VMEM((16,), jnp.int32),
          pltpu.SemaphoreType.DMA,
          pltpu.SemaphoreType.DMA,
          pltpu.SemaphoreType.DMA,
      ],
      name="fifo_enqueue",
  )


def kernel(storage, vals, pointer):
  capacity, dims = storage.shape
  n = vals.shape[0]
  ptr_vec = jnp.broadcast_to(jnp.asarray(pointer, jnp.int32), (16,))
  fifo = _make_fifo(capacity, n, dims)
  return fifo(storage, vals, ptr_vec)


# speedup vs baseline: 6.0682x; 6.0682x over previous
import functools

import jax
import jax.numpy as jnp
from jax import lax
from jax.experimental import pallas as pl
from jax.experimental.pallas import tpu as pltpu
from jax.experimental.pallas import tpu_sc as plsc
from jax._src.pallas import mpmd as _mpmd


@functools.lru_cache(maxsize=None)
def _make_fifo(capacity: int, n: int, dims: int):
  info = plsc.get_sparse_core_info()
  nc, ns, lanes = info.num_cores, info.num_subcores, info.num_lanes
  nw = nc * ns
  assert capacity % nw == 0 and n % ns == 0 and n % lanes == 0
  assert n <= capacity // nc
  slab = capacity // nw
  piece = n // ns
  nidx = piece // 128
  half = capacity // nc
  chunk = 256
  nbuf = 2
  nchunk = slab // chunk
  mesh = plsc.VectorSubcoreMesh(core_axis_name="c", subcore_axis_name="s")

  def body(storage_ref, vals_ref, ptr_ref, out_ref,
           didx_v, vidx_v, rows_v, ptr_v, buf_v, sem_cg, sem_cs, sem_g,
           sem_s):
    c = lax.axis_index("c")
    s = lax.axis_index("s")
    wid = c * ns + s
    base = wid * slab

    pltpu.sync_copy(ptr_ref, ptr_v)
    p = lax.rem(ptr_v[...][0], capacity)
    wrap = jnp.maximum(p + n - capacity, 0)
    a_end = p + n - wrap
    h0 = c * half
    h1 = h0 + half
    lo_a = jnp.maximum(p, h0)
    len_a = jnp.maximum(jnp.minimum(a_end, h1) - lo_a, 0)
    len_b = jnp.maximum(jnp.minimum(wrap, h1) - h0, 0)
    win_len = len_a + len_b
    lo = jnp.where(len_a > 0, lo_a, h0)
    vbase = lax.rem(lo - p + capacity, capacity)

    off = s * piece
    last = jnp.maximum(win_len - 1, 0)
    for j in range(nidx):
      for i in range(128 // lanes):
        m = jnp.minimum(
            off + j * 128 + i * lanes + lax.iota(jnp.int32, lanes), last)
        didx_v[j, pl.ds(i * lanes, lanes)] = lo + m
        vidx_v[j, pl.ds(i * lanes, lanes)] = jnp.minimum(vbase + m, n - 1)
    vg = [
        pltpu.async_copy(vals_ref.at[vidx_v.at[j]], rows_v.at[j], sem_g)
        for j in range(nidx)
    ]

    copies = {}
    for k in range(nchunk):
      if k == 0:
        copies["g", 0] = pltpu.async_copy(
            storage_ref.at[pl.ds(base, chunk)], buf_v.at[0], sem_cg)
      if k >= 1:
        copies["s", k - 1].wait()
      if k + 1 < nchunk:
        copies["g", k + 1] = pltpu.async_copy(
            storage_ref.at[pl.ds(base + (k + 1) * chunk, chunk)],
            buf_v.at[(k + 1) % nbuf], sem_cg)
      copies["g", k].wait()
      copies["s", k] = pltpu.async_copy(
          buf_v.at[k % nbuf], out_ref.at[pl.ds(base + k * chunk, chunk)],
          sem_cs)
    copies["s", nchunk - 1].wait()

    for d in vg:
      d.wait()
    plsc.subcore_barrier()

    @pl.when(off < win_len)
    def _scatter():
      for j in range(nidx):
        pltpu.async_copy(rows_v.at[j], out_ref.at[didx_v.at[j]], sem_s).wait()

  return _mpmd.mpmd_map(
      [(mesh, body)],
      out_types=jax.ShapeDtypeStruct((capacity, dims), jnp.float32),
      scratch_types=[
          pltpu.VMEM((nidx, 128), jnp.int32),
          pltpu.VMEM((nidx, 128), jnp.int32),
          pltpu.VMEM((nidx, 128, dims), jnp.float32),
          pltpu.VMEM((16,), jnp.int32),
          pltpu.VMEM((nbuf, chunk, dims), jnp.float32),
          pltpu.SemaphoreType.DMA,
          pltpu.SemaphoreType.DMA,
          pltpu.SemaphoreType.DMA,
          pltpu.SemaphoreType.DMA,
      ],
      name="fifo_enqueue",
  )


def kernel(storage, vals, pointer):
  capacity, dims = storage.shape
  n = vals.shape[0]
  ptr_vec = jnp.broadcast_to(jnp.asarray(pointer, jnp.int32), (16,))
  fifo = _make_fifo(capacity, n, dims)
  return fifo(storage, vals, ptr_vec)


# speedup vs baseline: 18.5672x; 3.0597x over previous
import functools

import jax
import jax.numpy as jnp
from jax import lax
from jax.experimental import pallas as pl
from jax.experimental.pallas import tpu as pltpu
from jax.experimental.pallas import tpu_sc as plsc
from jax._src.pallas import mpmd as _mpmd


@functools.lru_cache(maxsize=None)
def _make_copy(capacity: int, dims: int):
  blk = 2048
  assert capacity % blk == 0

  def body(x_ref, o_ref):
    o_ref[...] = x_ref[...]

  return pl.pallas_call(
      body,
      grid=(capacity // blk,),
      in_specs=[pl.BlockSpec((blk, dims), lambda i: (i, 0))],
      out_specs=pl.BlockSpec((blk, dims), lambda i: (i, 0)),
      out_shape=jax.ShapeDtypeStruct((capacity, dims), jnp.float32),
  )


@functools.lru_cache(maxsize=None)
def _make_scatter(capacity: int, n: int, dims: int):
  info = plsc.get_sparse_core_info()
  nc, ns, lanes = info.num_cores, info.num_subcores, info.num_lanes
  nw = nc * ns
  assert n % nw == 0, (n, nw)
  rows_per_w = n // nw
  assert rows_per_w % lanes == 0 and rows_per_w <= 128
  mesh = plsc.VectorSubcoreMesh(core_axis_name="c", subcore_axis_name="s")

  def body(storage_ref, vals_ref, ptr_ref, out_ref, idx_v, vals_v, ptr_v, sem):
    del storage_ref
    wid = lax.axis_index("s") * nc + lax.axis_index("c")
    base = wid * rows_per_w
    pltpu.sync_copy(vals_ref.at[pl.ds(base, rows_per_w)], vals_v)
    pltpu.sync_copy(ptr_ref, ptr_v)
    p = ptr_v[...]
    for i in range(rows_per_w // lanes):
      off = base + i * lanes
      idx_v[pl.ds(i * lanes, lanes)] = lax.rem(
          p + off + lax.iota(jnp.int32, lanes), capacity
      )
    pltpu.async_copy(vals_v, out_ref.at[idx_v], sem).wait()

  return _mpmd._mpmd_map(
      [(mesh, body)],
      out_types=jax.ShapeDtypeStruct((capacity, dims), jnp.float32),
      input_output_aliases={0: 0},
      scratch_types=[
          pltpu.VMEM((rows_per_w,), jnp.int32),
          pltpu.VMEM((rows_per_w, dims), jnp.float32),
          pltpu.VMEM((16,), jnp.int32),
          pltpu.SemaphoreType.DMA,
      ],
      name="fifo_scatter",
  )


def kernel(storage, vals, pointer):
  capacity, dims = storage.shape
  n = vals.shape[0]
  ptr_vec = jnp.broadcast_to(jnp.asarray(pointer, jnp.int32), (16,))
  copied = _make_copy(capacity, dims)(storage)
  return _make_scatter(capacity, n, dims)(copied, vals, ptr_vec)


# speedup vs baseline: 23.2638x; 1.2530x over previous
import functools

import jax
import jax.numpy as jnp
from jax import lax
from jax.experimental import pallas as pl
from jax.experimental.pallas import tpu as pltpu
from jax.experimental.pallas import tpu_sc as plsc
from jax._src.pallas import mpmd as _mpmd


@functools.lru_cache(maxsize=None)
def _make_copy(capacity: int, dims: int):
  blk = 8192
  assert capacity % blk == 0

  def body(x_ref, o_ref):
    o_ref[...] = x_ref[...]

  return pl.pallas_call(
      body,
      grid=(capacity // blk,),
      in_specs=[pl.BlockSpec((blk, dims), lambda i: (i, 0))],
      out_specs=pl.BlockSpec((blk, dims), lambda i: (i, 0)),
      out_shape=jax.ShapeDtypeStruct((capacity, dims), jnp.float32),
  )


@functools.lru_cache(maxsize=None)
def _make_scatter(capacity: int, n: int, dims: int):
  info = plsc.get_sparse_core_info()
  nc, ns, lanes = info.num_cores, info.num_subcores, info.num_lanes
  nw = nc * ns
  assert n % nw == 0, (n, nw)
  rows_per_w = n // nw
  assert rows_per_w % lanes == 0 and rows_per_w <= 128
  mesh = plsc.VectorSubcoreMesh(core_axis_name="c", subcore_axis_name="s")

  def body(storage_ref, vals_ref, ptr_ref, out_ref, idx_v, vals_v, ptr_v, sem):
    del storage_ref
    wid = lax.axis_index("s") * nc + lax.axis_index("c")
    base = wid * rows_per_w
    pltpu.sync_copy(vals_ref.at[pl.ds(base, rows_per_w)], vals_v)
    pltpu.sync_copy(ptr_ref, ptr_v)
    p = ptr_v[...]
    for i in range(rows_per_w // lanes):
      off = base + i * lanes
      idx_v[pl.ds(i * lanes, lanes)] = lax.rem(
          p + off + lax.iota(jnp.int32, lanes), capacity
      )
    pltpu.async_copy(vals_v, out_ref.at[idx_v], sem).wait()

  return _mpmd._mpmd_map(
      [(mesh, body)],
      out_types=jax.ShapeDtypeStruct((capacity, dims), jnp.float32),
      input_output_aliases={0: 0},
      scratch_types=[
          pltpu.VMEM((rows_per_w,), jnp.int32),
          pltpu.VMEM((rows_per_w, dims), jnp.float32),
          pltpu.VMEM((16,), jnp.int32),
          pltpu.SemaphoreType.DMA,
      ],
      name="fifo_scatter",
  )


def kernel(storage, vals, pointer):
  capacity, dims = storage.shape
  n = vals.shape[0]
  ptr_vec = jnp.broadcast_to(jnp.asarray(pointer, jnp.int32), (16,))
  copied = _make_copy(capacity, dims)(storage)
  return _make_scatter(capacity, n, dims)(copied, vals, ptr_vec)


# speedup vs baseline: 24.0966x; 1.0358x over previous
import functools

import jax
import jax.numpy as jnp
from jax import lax
from jax.experimental import pallas as pl
from jax.experimental.pallas import tpu as pltpu
from jax.experimental.pallas import tpu_sc as plsc
from jax._src.pallas import mpmd as _mpmd


@functools.lru_cache(maxsize=None)
def _make_copy(capacity: int, dims: int):
  blk = 8192
  assert capacity % blk == 0

  def body(x_ref, o_ref):
    o_ref[...] = x_ref[...]

  return pl.pallas_call(
      body,
      grid=(capacity // blk,),
      in_specs=[pl.BlockSpec((blk, dims), lambda i: (i, 0))],
      out_specs=pl.BlockSpec((blk, dims), lambda i: (i, 0)),
      out_shape=jax.ShapeDtypeStruct((capacity, dims), jnp.float32),
  )


@functools.lru_cache(maxsize=None)
def _make_scatter(capacity: int, n: int, dims: int):
  info = plsc.get_sparse_core_info()
  nc, ns, lanes = info.num_cores, info.num_subcores, info.num_lanes
  nw = nc * ns
  assert n % nw == 0, (n, nw)
  rows_per_w = n // nw
  assert rows_per_w % lanes == 0 and rows_per_w <= 128
  mesh = plsc.VectorSubcoreMesh(core_axis_name="c", subcore_axis_name="s")

  def body(storage_ref, vals_ref, ptr_ref, out_ref, idx_v, vals_v, ptr_v,
           sem_v, sem_p, sem_s):
    del storage_ref
    wid = lax.axis_index("s") * nc + lax.axis_index("c")
    base = wid * rows_per_w
    vals_dma = pltpu.async_copy(
        vals_ref.at[pl.ds(base, rows_per_w)], vals_v, sem_v)
    ptr_dma = pltpu.async_copy(ptr_ref, ptr_v, sem_p)
    ptr_dma.wait()
    p = ptr_v[...]
    for i in range(rows_per_w // lanes):
      off = base + i * lanes
      idx_v[pl.ds(i * lanes, lanes)] = lax.rem(
          p + off + lax.iota(jnp.int32, lanes), capacity
      )
    vals_dma.wait()
    pltpu.async_copy(vals_v, out_ref.at[idx_v], sem_s).wait()

  return _mpmd._mpmd_map(
      [(mesh, body)],
      out_types=jax.ShapeDtypeStruct((capacity, dims), jnp.float32),
      input_output_aliases={0: 0},
      scratch_types=[
          pltpu.VMEM((rows_per_w,), jnp.int32),
          pltpu.VMEM((rows_per_w, dims), jnp.float32),
          pltpu.VMEM((16,), jnp.int32),
          pltpu.SemaphoreType.DMA,
          pltpu.SemaphoreType.DMA,
          pltpu.SemaphoreType.DMA,
      ],
      name="fifo_scatter",
  )


def kernel(storage, vals, pointer):
  capacity, dims = storage.shape
  n = vals.shape[0]
  ptr_vec = jnp.broadcast_to(jnp.asarray(pointer, jnp.int32), (16,))
  copied = _make_copy(capacity, dims)(storage)
  return _make_scatter(capacity, n, dims)(copied, vals, ptr_vec)


# speedup vs baseline: 24.8594x; 1.0317x over previous
import functools

import jax
import jax.numpy as jnp
from jax import lax
from jax.experimental import pallas as pl
from jax.experimental.pallas import tpu as pltpu
from jax.experimental.pallas import tpu_sc as plsc
from jax._src.pallas import mpmd as _mpmd


@functools.lru_cache(maxsize=None)
def _make_copy(capacity: int, dims: int):
  blk = 16384
  assert capacity % blk == 0

  def body(x_ref, o_ref):
    o_ref[...] = x_ref[...]

  return pl.pallas_call(
      body,
      grid=(capacity // blk,),
      in_specs=[pl.BlockSpec((blk, dims), lambda i: (i, 0))],
      out_specs=pl.BlockSpec((blk, dims), lambda i: (i, 0)),
      out_shape=jax.ShapeDtypeStruct((capacity, dims), jnp.float32),
  )


@functools.lru_cache(maxsize=None)
def _make_scatter(capacity: int, n: int, dims: int):
  info = plsc.get_sparse_core_info()
  nc, ns, lanes = info.num_cores, info.num_subcores, info.num_lanes
  nw = nc * ns
  assert n % nw == 0, (n, nw)
  rows_per_w = n // nw
  assert rows_per_w % lanes == 0 and rows_per_w <= 128
  mesh = plsc.VectorSubcoreMesh(core_axis_name="c", subcore_axis_name="s")

  def body(storage_ref, vals_ref, ptr_ref, out_ref, idx_v, vals_v, ptr_v,
           sem_v, sem_p, sem_s):
    del storage_ref
    wid = lax.axis_index("s") * nc + lax.axis_index("c")
    base = wid * rows_per_w
    hw = rows_per_w // 2
    vals_dma = [
        pltpu.async_copy(
            vals_ref.at[pl.ds(base + h * hw, hw)],
            vals_v.at[pl.ds(h * hw, hw)], sem_v)
        for h in range(2)
    ]
    ptr_dma = pltpu.async_copy(ptr_ref, ptr_v, sem_p)
    ptr_dma.wait()
    p = ptr_v[...]
    for h in range(2):
      for i in range(hw // lanes):
        off = base + h * hw + i * lanes
        idx_v[h, pl.ds(i * lanes, lanes)] = lax.rem(
            p + off + lax.iota(jnp.int32, lanes), capacity
        )
    scat = []
    for h in range(2):
      vals_dma[h].wait()
      scat.append(
          pltpu.async_copy(
              vals_v.at[pl.ds(h * hw, hw)],
              out_ref.at[idx_v.at[h]], sem_s))
    for d in scat:
      d.wait()

  return _mpmd._mpmd_map(
      [(mesh, body)],
      out_types=jax.ShapeDtypeStruct((capacity, dims), jnp.float32),
      input_output_aliases={0: 0},
      scratch_types=[
          pltpu.VMEM((2, rows_per_w // 2), jnp.int32),
          pltpu.VMEM((rows_per_w, dims), jnp.float32),
          pltpu.VMEM((16,), jnp.int32),
          pltpu.SemaphoreType.DMA,
          pltpu.SemaphoreType.DMA,
          pltpu.SemaphoreType.DMA,
      ],
      name="fifo_scatter",
  )


def kernel(storage, vals, pointer):
  capacity, dims = storage.shape
  n = vals.shape[0]
  ptr_vec = jnp.broadcast_to(jnp.asarray(pointer, jnp.int32), (16,))
  copied = _make_copy(capacity, dims)(storage)
  return _make_scatter(capacity, n, dims)(copied, vals, ptr_vec)


# speedup vs baseline: 24.9974x; 1.0056x over previous
import functools

import jax
import jax.numpy as jnp
from jax import lax
from jax.experimental import pallas as pl
from jax.experimental.pallas import tpu as pltpu
from jax.experimental.pallas import tpu_sc as plsc
from jax._src.pallas import mpmd as _mpmd


@functools.lru_cache(maxsize=None)
def _make_copy(capacity: int, dims: int):
  blk = 16384
  assert capacity % blk == 0

  def body(x_ref, o_ref):
    o_ref[...] = x_ref[...]

  return pl.pallas_call(
      body,
      grid=(capacity // blk,),
      in_specs=[pl.BlockSpec((blk, dims), lambda i: (i, 0))],
      out_specs=pl.BlockSpec((blk, dims), lambda i: (i, 0)),
      out_shape=jax.ShapeDtypeStruct((capacity, dims), jnp.float32),
  )


@functools.lru_cache(maxsize=None)
def _make_scatter(capacity: int, n: int, dims: int):
  info = plsc.get_sparse_core_info()
  nc, ns, lanes = info.num_cores, info.num_subcores, info.num_lanes
  nw = nc * ns
  assert n % nw == 0, (n, nw)
  rows_per_w = n // nw
  assert rows_per_w % lanes == 0 and rows_per_w <= 128
  mesh = plsc.VectorSubcoreMesh(core_axis_name="c", subcore_axis_name="s")

  def body(storage_ref, vals_ref, ptr_ref, out_ref, idx_v, vals_v, ptr_v,
           sem_v, sem_p, sem_s):
    del storage_ref
    wid = lax.axis_index("s") * nc + lax.axis_index("c")
    base = wid * rows_per_w
    npipe = 4
    hw = rows_per_w // npipe
    vals_dma = [
        pltpu.async_copy(
            vals_ref.at[pl.ds(base + h * hw, hw)],
            vals_v.at[pl.ds(h * hw, hw)], sem_v)
        for h in range(npipe)
    ]
    ptr_dma = pltpu.async_copy(ptr_ref, ptr_v, sem_p)
    ptr_dma.wait()
    p = ptr_v[...]
    for h in range(npipe):
      for i in range(hw // lanes):
        off = base + h * hw + i * lanes
        idx_v[h, pl.ds(i * lanes, lanes)] = lax.rem(
            p + off + lax.iota(jnp.int32, lanes), capacity
        )
    scat = []
    for h in range(npipe):
      vals_dma[h].wait()
      scat.append(
          pltpu.async_copy(
              vals_v.at[pl.ds(h * hw, hw)],
              out_ref.at[idx_v.at[h]], sem_s))
    for d in scat:
      d.wait()

  return _mpmd._mpmd_map(
      [(mesh, body)],
      out_types=jax.ShapeDtypeStruct((capacity, dims), jnp.float32),
      input_output_aliases={0: 0},
      scratch_types=[
          pltpu.VMEM((4, rows_per_w // 4), jnp.int32),
          pltpu.VMEM((rows_per_w, dims), jnp.float32),
          pltpu.VMEM((16,), jnp.int32),
          pltpu.SemaphoreType.DMA,
          pltpu.SemaphoreType.DMA,
          pltpu.SemaphoreType.DMA,
      ],
      name="fifo_scatter",
  )


def kernel(storage, vals, pointer):
  capacity, dims = storage.shape
  n = vals.shape[0]
  ptr_vec = jnp.broadcast_to(jnp.asarray(pointer, jnp.int32), (16,))
  copied = _make_copy(capacity, dims)(storage)
  return _make_scatter(capacity, n, dims)(copied, vals, ptr_vec)
